# baseline (speedup 1.0000x reference)
import jax
import jax.numpy as jnp
from jax import lax
from jax.experimental import pallas as pl
from jax.experimental.pallas import tpu as pltpu
from jax.experimental.pallas import tpu_sc as plsc

f32 = jnp.float32
i32 = jnp.int32

EMB = 100000
N = 1024 * 20
D = 64
C, I = 10, 5
NC, NS = 2, 16
NW = NC * NS
PPW = N // NW
P = 16
NCH = PPW // P

_MESH = plsc.VectorSubcoreMesh(core_axis_name="c", subcore_axis_name="s")


def _sc_body(wout, win, pu_h, pv_h, iv_h, s1_h, s2_h,
             pu_v, pv_v, iv_v, ub0, cb0, ob0, ib0, ub1, cb1, ob1, ib1,
             s1_v, s2_v, sem0, sem1):
    wid = lax.axis_index("s") * NC + lax.axis_index("c")

    pltpu.sync_copy(pu_h.at[pl.ds(wid * PPW, PPW)], pu_v)
    pltpu.sync_copy(pv_h.at[pl.ds(wid * PPW * C, PPW * C)], pv_v)
    pltpu.sync_copy(iv_h.at[pl.ds(wid * PPW * I, PPW * I)], iv_v)

    def remap(ref, n):
        @pl.loop(0, n, step=16)
        def _(j):
            v = ref[pl.ds(j, 16)]
            hi = jnp.bitwise_and(v, ~(_RB - 1))
            lo = jnp.bitwise_and(v, _RH - 1)
            hb = jnp.bitwise_and(jnp.right_shift(v, 13), 1)
            ref[pl.ds(j, 16)] = hi + lo + lo + hb

    remap(pu_v, PPW)
    remap(pv_v, PPW * C)
    remap(iv_v, PPW * I)

    def issue(k, ub, cb, ob, ib, sem):
        pltpu.async_copy(wout.at[pu_v.at[pl.ds(k * P, P)]], ub, sem)
        pltpu.async_copy(win.at[pv_v.at[pl.ds(k * P * C, 80)]],
                         cb.at[pl.ds(0, 80)], sem)
        pltpu.async_copy(win.at[pv_v.at[pl.ds(k * P * C + 80, 80)]],
                         cb.at[pl.ds(80, 80)], sem)
        pltpu.async_copy(wout.at[pv_v.at[pl.ds(k * P * C, 80)]],
                         ob.at[pl.ds(0, 80)], sem)
        pltpu.async_copy(wout.at[pv_v.at[pl.ds(k * P * C + 80, 80)]],
                         ob.at[pl.ds(80, 80)], sem)
        pltpu.async_copy(win.at[iv_v.at[pl.ds(k * P * I, P * I)]], ib, sem)

    def drain(ub, cb, ob, ib, sem):
        pltpu.make_async_copy(wout.at[pu_v.at[pl.ds(0, P)]], ub, sem).wait()
        pltpu.make_async_copy(win.at[pv_v.at[pl.ds(0, 80)]],
                              cb.at[pl.ds(0, 80)], sem).wait()
        pltpu.make_async_copy(win.at[pv_v.at[pl.ds(0, 80)]],
                              cb.at[pl.ds(80, 80)], sem).wait()
        pltpu.make_async_copy(wout.at[pv_v.at[pl.ds(0, 80)]],
                              ob.at[pl.ds(0, 80)], sem).wait()
        pltpu.make_async_copy(wout.at[pv_v.at[pl.ds(0, 80)]],
                              ob.at[pl.ds(80, 80)], sem).wait()
        pltpu.make_async_copy(win.at[iv_v.at[pl.ds(0, P * I)]], ib, sem).wait()

    p16 = lax.iota(i32, 16)
    base_c = p16 * C
    base_i = p16 * I

    def compute(k, ub, cb, ob, ib):
        off = k * P

        def score_body(dd, acc):
            colv0 = jnp.bitwise_and(p16 + 2 * dd, 63)
            colv1 = jnp.bitwise_and(colv0 + 1, 63)
            u0 = plsc.load_gather(ub, [p16, colv0])
            u1 = plsc.load_gather(ub, [p16, colv1])
            return tuple(
                acc[c]
                + u0 * plsc.load_gather(cb, [base_c + c, colv0])
                + u1 * plsc.load_gather(cb, [base_c + c, colv1])
                for c in range(C))

        accs = lax.fori_loop(0, D // 2, score_body,
                             tuple(jnp.zeros((16,), f32) for _ in range(C)))
        for c in range(C):
            s1_v[c, pl.ds(off, P)] = accs[c]

        for ch in range(2):
            def info_body(dd, acc, ch=ch):
                colv0 = jnp.bitwise_and(p16 + 2 * dd, 63)
                colv1 = jnp.bitwise_and(colv0 + 1, 63)
                acc = list(acc)
                for colv in (colv0, colv1):
                    infs = [plsc.load_gather(ib, [base_i + i, colv])
                            for i in range(I)]
                    for cc in range(5):
                        c = ch * 5 + cc
                        co = plsc.load_gather(ob, [base_c + c, colv])
                        for i in range(I):
                            acc[cc * I + i] = acc[cc * I + i] + co * infs[i]
                return tuple(acc)

            acc2 = lax.fori_loop(0, D // 2, info_body,
                                 tuple(jnp.zeros((16,), f32) for _ in range(25)))
            for cc in range(5):
                c = ch * 5 + cc
                for i in range(I):
                    s2_v[c * I + i, pl.ds(off, P)] = acc2[cc * I + i]

    issue(0, ub0, cb0, ob0, ib0, sem0)

    @pl.loop(0, NCH, step=2)
    def _(k):
        issue(k + 1, ub1, cb1, ob1, ib1, sem1)
        drain(ub0, cb0, ob0, ib0, sem0)
        compute(k, ub0, cb0, ob0, ib0)

        @pl.when(k + 2 < NCH)
        def _():
            issue(k + 2, ub0, cb0, ob0, ib0, sem0)

        drain(ub1, cb1, ob1, ib1, sem1)
        compute(k + 1, ub1, cb1, ob1, ib1)

    pltpu.sync_copy(s1_v, s1_h.at[wid])
    pltpu.sync_copy(s2_v, s2_h.at[wid])


_sc_scores = pl.kernel(
    _sc_body,
    out_type=(jax.ShapeDtypeStruct((NW, C, PPW), f32),
              jax.ShapeDtypeStruct((NW, C * I, PPW), f32)),
    mesh=_MESH,
    compiler_params=pltpu.CompilerParams(needs_layout_passes=False,
                                         use_tc_tiling_on_sc=False),
    scratch_types=[
        pltpu.VMEM((PPW,), i32),
        pltpu.VMEM((PPW * C,), i32),
        pltpu.VMEM((PPW * I,), i32),
        pltpu.VMEM((P, D), f32),
        pltpu.VMEM((P * C, D), f32),
        pltpu.VMEM((P * C, D), f32),
        pltpu.VMEM((P * I, D), f32),
        pltpu.VMEM((P, D), f32),
        pltpu.VMEM((P * C, D), f32),
        pltpu.VMEM((P * C, D), f32),
        pltpu.VMEM((P * I, D), f32),
        pltpu.VMEM((C, PPW), f32),
        pltpu.VMEM((C * I, PPW), f32),
        pltpu.SemaphoreType.DMA,
        pltpu.SemaphoreType.DMA,
    ],
)


_RB = 16384
_RH = _RB // 2
_RG = (EMB + _RB - 1) // _RB
EMB_P = _RG * _RB


def _repack_body(a_ref, b_ref, oa_ref, ob_ref):
    xa = a_ref[...]
    oa_ref[...] = jnp.concatenate([xa[:, :_RH].T, xa[:, _RH:].T], axis=1)
    xb = b_ref[...]
    ob_ref[...] = jnp.concatenate([xb[:, :_RH].T, xb[:, _RH:].T], axis=1)


def _repack(wout_t, win_t):
    return pl.pallas_call(
        _repack_body,
        grid=(_RG,),
        in_specs=[pl.BlockSpec((D, _RB), lambda i: (0, i)),
                  pl.BlockSpec((D, _RB), lambda i: (0, i))],
        out_specs=[pl.BlockSpec((_RH, 128), lambda i: (i, 0)),
                   pl.BlockSpec((_RH, 128), lambda i: (i, 0))],
        out_shape=(jax.ShapeDtypeStruct((EMB_P // 2, 128), f32),
                   jax.ShapeDtypeStruct((EMB_P // 2, 128), f32)),
    )(wout_t, win_t)


def _loss_body(s1_ref, s2_ref, m1_ref, ms_ref, mr_ref, o_ref):
    x1 = jnp.clip(s1_ref[...] * m1_ref[...], -10.0, 10.0)
    l1 = jnp.sum(-jax.nn.log_sigmoid(x1))
    x2 = jnp.clip(s2_ref[...], -10.0, 10.0) * ms_ref[...]
    l2 = jnp.sum(mr_ref[...] * (-jax.nn.log_sigmoid(x2)))
    o_ref[...] = jnp.reshape(l1 + l2, (1, 1))


def kernel(pos_u, pos_v, info_v, W_out, W_in, context_mask, sig_mask, score_mask):
    pu = pos_u.reshape(N).astype(i32)
    pv = pos_v.reshape(N * C).astype(i32)
    iv = info_v.reshape(N * I).astype(i32)

    wo_lin, wi_lin = _repack(W_out.astype(f32).T, W_in.astype(f32).T)
    s1, s2 = _sc_scores(wo_lin.reshape(EMB_P, D), wi_lin.reshape(EMB_P, D),
                        pu, pv, iv)

    m1 = jnp.tile(context_mask.astype(f32), NW).reshape(NW * C, 1)
    ms = jnp.tile(sig_mask.astype(f32), NW * C).reshape(NW * C * I, 1)
    mr = jnp.tile(score_mask.astype(f32), NW * C).reshape(NW * C * I, 1)

    out = pl.pallas_call(
        _loss_body,
        out_shape=jax.ShapeDtypeStruct((1, 1), f32),
    )(s1.reshape(NW * C, PPW), s2.reshape(NW * C * I, PPW), m1, ms, mr)
    return out[0, 0]


# speedup vs baseline: 1.0476x; 1.0476x over previous
import jax
import jax.numpy as jnp
from jax import lax
from jax.experimental import pallas as pl
from jax.experimental.pallas import tpu as pltpu
from jax.experimental.pallas import tpu_sc as plsc

f32 = jnp.float32
i32 = jnp.int32

EMB = 100000
N = 1024 * 20
D = 64
C, I = 10, 5
NC, NS = 2, 16
NW = NC * NS
PPW = N // NW
P = 16
NCH = PPW // P

_MESH = plsc.VectorSubcoreMesh(core_axis_name="c", subcore_axis_name="s")


def _sc_body(wout, win, pu_h, pv_h, iv_h, s1_h, s2_h,
             pu_v, pv_v, iv_v, ub0, cb0, ob0, ib0, ub1, cb1, ob1, ib1,
             s1_v, s2_v, sem0, sem1, sem2, sem3):
    wid = lax.axis_index("s") * NC + lax.axis_index("c")

    pltpu.sync_copy(pu_h.at[pl.ds(wid * PPW, PPW)], pu_v)
    pltpu.sync_copy(pv_h.at[pl.ds(wid * PPW * C, PPW * C)], pv_v)
    pltpu.sync_copy(iv_h.at[pl.ds(wid * PPW * I, PPW * I)], iv_v)

    def remap(ref, n):
        @pl.loop(0, n, step=16)
        def _(j):
            v = ref[pl.ds(j, 16)]
            hi = jnp.bitwise_and(v, ~(_RB - 1))
            lo = jnp.bitwise_and(v, _RH - 1)
            hb = jnp.bitwise_and(jnp.right_shift(v, 13), 1)
            ref[pl.ds(j, 16)] = hi + lo + lo + hb

    remap(pu_v, PPW)
    remap(pv_v, PPW * C)
    remap(iv_v, PPW * I)

    def issue(k, ub, cb, ob, ib, sem_s, sem_i):
        pltpu.async_copy(wout.at[pu_v.at[pl.ds(k * P, P)]], ub, sem_s)
        pltpu.async_copy(win.at[pv_v.at[pl.ds(k * P * C, 80)]],
                         cb.at[pl.ds(0, 80)], sem_s)
        pltpu.async_copy(win.at[pv_v.at[pl.ds(k * P * C + 80, 80)]],
                         cb.at[pl.ds(80, 80)], sem_s)
        pltpu.async_copy(wout.at[pv_v.at[pl.ds(k * P * C, 80)]],
                         ob.at[pl.ds(0, 80)], sem_i)
        pltpu.async_copy(wout.at[pv_v.at[pl.ds(k * P * C + 80, 80)]],
                         ob.at[pl.ds(80, 80)], sem_i)
        pltpu.async_copy(win.at[iv_v.at[pl.ds(k * P * I, P * I)]], ib, sem_i)

    def drain_s(ub, cb, sem_s):
        pltpu.make_async_copy(wout.at[pu_v.at[pl.ds(0, P)]], ub, sem_s).wait()
        pltpu.make_async_copy(win.at[pv_v.at[pl.ds(0, 80)]],
                              cb.at[pl.ds(0, 80)], sem_s).wait()
        pltpu.make_async_copy(win.at[pv_v.at[pl.ds(0, 80)]],
                              cb.at[pl.ds(80, 80)], sem_s).wait()

    def drain_i(ob, ib, sem_i):
        pltpu.make_async_copy(wout.at[pv_v.at[pl.ds(0, 80)]],
                              ob.at[pl.ds(0, 80)], sem_i).wait()
        pltpu.make_async_copy(wout.at[pv_v.at[pl.ds(0, 80)]],
                              ob.at[pl.ds(80, 80)], sem_i).wait()
        pltpu.make_async_copy(win.at[iv_v.at[pl.ds(0, P * I)]], ib, sem_i).wait()

    p16 = lax.iota(i32, 16)
    base_c = p16 * C
    base_i = p16 * I

    def score_pass(k, ub, cb):
        off = k * P

        def score_body(d, acc):
            colv = jnp.bitwise_and(p16 + d, 63)
            u = plsc.load_gather(ub, [p16, colv])
            return tuple(
                acc[c] + u * plsc.load_gather(cb, [base_c + c, colv])
                for c in range(C))

        accs = lax.fori_loop(0, D, score_body,
                             tuple(jnp.zeros((16,), f32) for _ in range(C)))
        for c in range(C):
            s1_v[c, pl.ds(off, P)] = accs[c]

    def info_pass(k, ob, ib):
        off = k * P
        for ch in range(2):
            def info_body(d, acc, ch=ch):
                colv = jnp.bitwise_and(p16 + d, 63)
                infs = [plsc.load_gather(ib, [base_i + i, colv])
                        for i in range(I)]
                acc = list(acc)
                for cc in range(5):
                    c = ch * 5 + cc
                    co = plsc.load_gather(ob, [base_c + c, colv])
                    for i in range(I):
                        acc[cc * I + i] = acc[cc * I + i] + co * infs[i]
                return tuple(acc)

            acc2 = lax.fori_loop(0, D, info_body,
                                 tuple(jnp.zeros((16,), f32) for _ in range(25)))
            for cc in range(5):
                c = ch * 5 + cc
                for i in range(I):
                    s2_v[c * I + i, pl.ds(off, P)] = acc2[cc * I + i]

    issue(0, ub0, cb0, ob0, ib0, sem0, sem1)

    @pl.loop(0, NCH, step=2)
    def _(k):
        issue(k + 1, ub1, cb1, ob1, ib1, sem2, sem3)
        drain_s(ub0, cb0, sem0)
        score_pass(k, ub0, cb0)
        drain_i(ob0, ib0, sem1)
        info_pass(k, ob0, ib0)

        @pl.when(k + 2 < NCH)
        def _():
            issue(k + 2, ub0, cb0, ob0, ib0, sem0, sem1)

        drain_s(ub1, cb1, sem2)
        score_pass(k + 1, ub1, cb1)
        drain_i(ob1, ib1, sem3)
        info_pass(k + 1, ob1, ib1)

    pltpu.sync_copy(s1_v, s1_h.at[wid])
    pltpu.sync_copy(s2_v, s2_h.at[wid])


_sc_scores = pl.kernel(
    _sc_body,
    out_type=(jax.ShapeDtypeStruct((NW, C, PPW), f32),
              jax.ShapeDtypeStruct((NW, C * I, PPW), f32)),
    mesh=_MESH,
    compiler_params=pltpu.CompilerParams(needs_layout_passes=False,
                                         use_tc_tiling_on_sc=False),
    scratch_types=[
        pltpu.VMEM((PPW,), i32),
        pltpu.VMEM((PPW * C,), i32),
        pltpu.VMEM((PPW * I,), i32),
        pltpu.VMEM((P, D), f32),
        pltpu.VMEM((P * C, D), f32),
        pltpu.VMEM((P * C, D), f32),
        pltpu.VMEM((P * I, D), f32),
        pltpu.VMEM((P, D), f32),
        pltpu.VMEM((P * C, D), f32),
        pltpu.VMEM((P * C, D), f32),
        pltpu.VMEM((P * I, D), f32),
        pltpu.VMEM((C, PPW), f32),
        pltpu.VMEM((C * I, PPW), f32),
        pltpu.SemaphoreType.DMA,
        pltpu.SemaphoreType.DMA,
        pltpu.SemaphoreType.DMA,
        pltpu.SemaphoreType.DMA,
    ],
)


_RB = 16384
_RH = _RB // 2
_RG = (EMB + _RB - 1) // _RB
EMB_P = _RG * _RB


def _repack_body(a_ref, b_ref, oa_ref, ob_ref):
    xa = a_ref[...]
    oa_ref[...] = jnp.concatenate([xa[:, :_RH].T, xa[:, _RH:].T], axis=1)
    xb = b_ref[...]
    ob_ref[...] = jnp.concatenate([xb[:, :_RH].T, xb[:, _RH:].T], axis=1)


def _repack(wout_t, win_t):
    return pl.pallas_call(
        _repack_body,
        grid=(_RG,),
        in_specs=[pl.BlockSpec((D, _RB), lambda i: (0, i)),
                  pl.BlockSpec((D, _RB), lambda i: (0, i))],
        out_specs=[pl.BlockSpec((_RH, 128), lambda i: (i, 0)),
                   pl.BlockSpec((_RH, 128), lambda i: (i, 0))],
        out_shape=(jax.ShapeDtypeStruct((EMB_P // 2, 128), f32),
                   jax.ShapeDtypeStruct((EMB_P // 2, 128), f32)),
    )(wout_t, win_t)


def _loss_body(s1_ref, s2_ref, m1_ref, ms_ref, mr_ref, o_ref):
    x1 = jnp.clip(s1_ref[...] * m1_ref[...], -10.0, 10.0)
    l1 = jnp.sum(-jax.nn.log_sigmoid(x1))
    x2 = jnp.clip(s2_ref[...], -10.0, 10.0) * ms_ref[...]
    l2 = jnp.sum(mr_ref[...] * (-jax.nn.log_sigmoid(x2)))
    o_ref[...] = jnp.reshape(l1 + l2, (1, 1))


def kernel(pos_u, pos_v, info_v, W_out, W_in, context_mask, sig_mask, score_mask):
    pu = pos_u.reshape(N).astype(i32)
    pv = pos_v.reshape(N * C).astype(i32)
    iv = info_v.reshape(N * I).astype(i32)

    wo_lin, wi_lin = _repack(W_out.astype(f32).T, W_in.astype(f32).T)
    s1, s2 = _sc_scores(wo_lin.reshape(EMB_P, D), wi_lin.reshape(EMB_P, D),
                        pu, pv, iv)

    m1 = jnp.tile(context_mask.astype(f32), NW).reshape(NW * C, 1)
    ms = jnp.tile(sig_mask.astype(f32), NW * C).reshape(NW * C * I, 1)
    mr = jnp.tile(score_mask.astype(f32), NW * C).reshape(NW * C * I, 1)

    out = pl.pallas_call(
        _loss_body,
        out_shape=jax.ShapeDtypeStruct((1, 1), f32),
    )(s1.reshape(NW * C, PPW), s2.reshape(NW * C * I, PPW), m1, ms, mr)
    return out[0, 0]
